# baseline (speedup 1.0000x reference)
import functools

import jax
import jax.numpy as jnp
from jax import lax
from jax.experimental import pallas as pl
from jax.experimental.pallas import tpu as pltpu
from jax.experimental.pallas import tpu_sc as plsc

LEN_USERS = 100000
EMBED_DIM = 128
BATCH = 16384

NUM_CORES = 2
NUM_SUBCORES = 16
LANES = 16
NUM_WORKERS = NUM_CORES * NUM_SUBCORES
BPW = BATCH // NUM_WORKERS
IDX_CHUNK = 128
NCHUNK = BPW // IDX_CHUNK
NGROUP = BPW // LANES
CVEC = EMBED_DIM // LANES


def _sc_kernel_body(uid_hbm, mov_hbm, table_hbm, w_hbm, b_hbm, out_hbm,
                    idx_v, rows_v, mov_v, acc_v, wb_v,
                    sem_i, sem_m, sem_w, sem_g0, sem_g1, sem_g2, sem_g3):
    wid = lax.axis_index("s") * NUM_CORES + lax.axis_index("c")
    base = wid * BPW

    cp_i = pltpu.async_copy(uid_hbm.at[pl.ds(base, BPW)], idx_v, sem_i)
    cp_m = pltpu.async_copy(mov_hbm.at[pl.ds(base, BPW)], mov_v, sem_m)
    cp_w = pltpu.async_copy(w_hbm, wb_v.at[pl.ds(0, 1)], sem_w)
    cp_b = pltpu.async_copy(b_hbm, wb_v.at[pl.ds(8, 1)], sem_w)
    cp_i.wait()

    sems = [sem_g0, sem_g1, sem_g2, sem_g3]
    copies = [
        pltpu.async_copy(table_hbm.at[idx_v.at[pl.ds(j * IDX_CHUNK, IDX_CHUNK)]],
                         rows_v.at[pl.ds(j * IDX_CHUNK, IDX_CHUNK)], sems[j])
        for j in range(NCHUNK)
    ]

    cp_m.wait()
    cp_w.wait()
    cp_b.wait()
    lane = lax.iota(jnp.int32, LANES)
    zeros = jnp.zeros((LANES,), jnp.int32)
    wb_vec = wb_v[...]
    w_splat = wb_vec.at[zeros].get(mode="promise_in_bounds")
    b_splat = wb_vec.at[zeros + 8].get(mode="promise_in_bounds")

    one = jnp.ones((LANES,), jnp.float32)

    mask15 = lane == (LANES - 1)

    for j in range(NCHUNK):
        copies[j].wait()

        @plsc.parallel_loop(j * IDX_CHUNK, (j + 1) * IDX_CHUNK)
        def sum_body(r):
            v = [rows_v[r, pl.ds(c * LANES, LANES)] for c in range(CVEC)]
            acc = ((v[0] + v[1]) + (v[2] + v[3])) + \
                  ((v[4] + v[5]) + (v[6] + v[7]))
            s = plsc.cumsum(acc)
            plsc.store_scatter(idx_v, [zeros + r],
                               plsc.bitcast(s, jnp.int32), mask=mask15)

    @plsc.parallel_loop(0, NGROUP)
    def epi_body(g):
        row0 = g * LANES
        res = plsc.bitcast(idx_v[pl.ds(row0, LANES)], jnp.float32)
        z = res * mov_v[pl.ds(row0, LANES)] * w_splat + b_splat
        acc_v[pl.ds(row0, LANES)] = one / (one + jnp.exp(-z))

    pltpu.sync_copy(acc_v, out_hbm.at[pl.ds(base, BPW)])


@jax.jit
def kernel(x, u_table, W, b):
    uid = x[:, 0]
    mov = x[:, 1].astype(jnp.float32)

    mesh = plsc.VectorSubcoreMesh(core_axis_name="c", subcore_axis_name="s",
                                  num_cores=NUM_CORES,
                                  num_subcores=NUM_SUBCORES)
    run = functools.partial(
        pl.kernel,
        out_type=jax.ShapeDtypeStruct((BATCH,), jnp.float32),
        mesh=mesh,
        compiler_params=pltpu.CompilerParams(needs_layout_passes=False),
        scratch_types=[
            pltpu.VMEM((BPW,), jnp.int32),
            pltpu.VMEM((BPW, EMBED_DIM), jnp.float32),
            pltpu.VMEM((BPW,), jnp.float32),
            pltpu.VMEM((BPW,), jnp.float32),
            pltpu.VMEM((LANES,), jnp.float32),
        ] + [pltpu.SemaphoreType.DMA] * 7,
    )(_sc_kernel_body)
    out = run(uid, mov, u_table, W.reshape(1), b)
    return out.reshape(BATCH, 1)


# speedup vs baseline: 1.0268x; 1.0268x over previous
import functools

import jax
import jax.numpy as jnp
from jax import lax
from jax.experimental import pallas as pl
from jax.experimental.pallas import tpu as pltpu
from jax.experimental.pallas import tpu_sc as plsc

LEN_USERS = 100000
EMBED_DIM = 128
BATCH = 16384

NUM_CORES = 2
NUM_SUBCORES = 16
LANES = 16
NUM_WORKERS = NUM_CORES * NUM_SUBCORES
BPW = BATCH // NUM_WORKERS
IDX_CHUNK = 128
NCHUNK = BPW // IDX_CHUNK
NGROUP = BPW // LANES
CVEC = EMBED_DIM // LANES


def _sc_kernel_body(uid_hbm, mov_hbm, table_hbm, w_hbm, b_hbm, out_hbm,
                    idx_v, rows_v, mov_v, acc_v, wb_v,
                    sem_i, sem_m, sem_w, sem_g):
    wid = lax.axis_index("s") * NUM_CORES + lax.axis_index("c")
    base = wid * BPW

    cp_i = pltpu.async_copy(uid_hbm.at[pl.ds(base, BPW)], idx_v, sem_i)
    cp_m = pltpu.async_copy(mov_hbm.at[pl.ds(base, BPW)], mov_v, sem_m)
    cp_w = pltpu.async_copy(w_hbm, wb_v.at[pl.ds(0, 1)], sem_w)
    cp_b = pltpu.async_copy(b_hbm, wb_v.at[pl.ds(8, 1)], sem_w)
    cp_i.wait()

    copies = [
        pltpu.async_copy(table_hbm.at[idx_v.at[pl.ds(j * IDX_CHUNK, IDX_CHUNK)]],
                         rows_v.at[pl.ds(j * IDX_CHUNK, IDX_CHUNK)], sem_g)
        for j in range(NCHUNK)
    ]

    cp_m.wait()
    cp_w.wait()
    cp_b.wait()
    lane = lax.iota(jnp.int32, LANES)
    zeros = jnp.zeros((LANES,), jnp.int32)
    wb_vec = wb_v[...]
    w_splat = wb_vec.at[zeros].get(mode="promise_in_bounds")
    b_splat = wb_vec.at[zeros + 8].get(mode="promise_in_bounds")

    one = jnp.ones((LANES,), jnp.float32)

    mask15 = lane == (LANES - 1)

    for cp in copies:
        cp.wait()

    @plsc.parallel_loop(0, BPW)
    def sum_body(r):
        v = [rows_v[r, pl.ds(c * LANES, LANES)] for c in range(CVEC)]
        acc = ((v[0] + v[1]) + (v[2] + v[3])) + \
              ((v[4] + v[5]) + (v[6] + v[7]))
        s = plsc.cumsum(acc)
        plsc.store_scatter(idx_v, [zeros + r],
                           plsc.bitcast(s, jnp.int32), mask=mask15)

    @plsc.parallel_loop(0, NGROUP)
    def epi_body(g):
        row0 = g * LANES
        res = plsc.bitcast(idx_v[pl.ds(row0, LANES)], jnp.float32)
        z = res * mov_v[pl.ds(row0, LANES)] * w_splat + b_splat
        acc_v[pl.ds(row0, LANES)] = one / (one + jnp.exp(-z))

    pltpu.sync_copy(acc_v, out_hbm.at[pl.ds(base, BPW)])


@jax.jit
def kernel(x, u_table, W, b):
    uid = x[:, 0]
    mov = x[:, 1].astype(jnp.float32)

    mesh = plsc.VectorSubcoreMesh(core_axis_name="c", subcore_axis_name="s",
                                  num_cores=NUM_CORES,
                                  num_subcores=NUM_SUBCORES)
    run = functools.partial(
        pl.kernel,
        out_type=jax.ShapeDtypeStruct((BATCH,), jnp.float32),
        mesh=mesh,
        compiler_params=pltpu.CompilerParams(needs_layout_passes=False),
        scratch_types=[
            pltpu.VMEM((BPW,), jnp.int32),
            pltpu.VMEM((BPW, EMBED_DIM), jnp.float32),
            pltpu.VMEM((BPW,), jnp.float32),
            pltpu.VMEM((BPW,), jnp.float32),
            pltpu.VMEM((LANES,), jnp.float32),
        ] + [pltpu.SemaphoreType.DMA] * 4,
    )(_sc_kernel_body)
    out = run(uid, mov, u_table, W.reshape(1), b)
    return out.reshape(BATCH, 1)
